# baseline (speedup 1.0000x reference)
import functools

import jax
import jax.numpy as jnp
import numpy as np
from jax import lax
from jax.experimental import pallas as pl
from jax.experimental.pallas import tpu as pltpu
from jax.experimental.pallas import tpu_sc as plsc

_N = 10000
_D = 128
_E = 320000
_NC = 2
_NS = 16
_NW = _NC * _NS
_PER_W = _E // _NW
_C = 80
_CHUNKS = _PER_W // _C
_G = _C // 16
_K = _D // 16


def _hr_body(h_ref, r_ref, o_ref):
    o_ref[:, :] = h_ref[:, :] * r_ref[:, :]


def _weight_rows(h, r):
    return pl.pallas_call(
        _hr_body,
        out_shape=jax.ShapeDtypeStruct((_N, _D), jnp.float32),
    )(h, r.reshape(1, _D))


_BITREV = (0, 8, 4, 12, 2, 10, 6, 14, 1, 9, 5, 13, 3, 11, 7, 15)
_LANES = np.arange(16, dtype=np.int32)


def _edge_dot_body(hr_hbm, h_hbm, src_hbm, dst_hbm, out_hbm,
                   sidx, didx, srows0, drows0, srows1, drows1,
                   srows2, drows2, qbuf, obuf, sem0, sem1, sem2):
    wid = lax.axis_index("s") * _NC + lax.axis_index("c")
    base = wid * _PER_W
    pltpu.sync_copy(src_hbm.at[pl.ds(base, _PER_W)], sidx)
    pltpu.sync_copy(dst_hbm.at[pl.ds(base, _PER_W)], didx)

    def fire(off, sbuf, dbuf, sem):
        pltpu.async_copy(hr_hbm.at[sidx.at[pl.ds(off, _C)]], sbuf, sem)
        pltpu.async_copy(h_hbm.at[didx.at[pl.ds(off, _C)]], dbuf, sem)

    def drain(sbuf, dbuf, sem):
        pltpu.make_async_copy(hr_hbm.at[sidx.at[pl.ds(0, _C)]],
                              sbuf, sem).wait()
        pltpu.make_async_copy(h_hbm.at[didx.at[pl.ds(0, _C)]],
                              dbuf, sem).wait()

    lane = lax.iota(jnp.int32, 16)

    def tree_sum(vs):
        while len(vs) > 1:
            odd = [vs[-1]] if len(vs) % 2 else []
            vs = [vs[i] + vs[i + 1] for i in range(0, len(vs) - 1, 2)] + odd
        return vs[0]

    def compute(coff, sbuf, dbuf):
        def group_body(g, carry):
            e0 = g * 16
            for j in range(16):
                e = e0 + j
                prods = [sbuf[e, pl.ds(k * 16, 16)] * dbuf[e, pl.ds(k * 16, 16)]
                         for k in range(_K)]
                qbuf[j, :] = tree_sum(prods)
            cols = [plsc.load_gather(qbuf, [lane, jnp.full((16,), l, jnp.int32)])
                    for l in range(16)]
            obuf[pl.ds(pl.multiple_of(coff + e0, 16), 16)] = tree_sum(cols)
            return carry
        lax.fori_loop(0, _G, group_body, 0)

    bufs = ((srows0, drows0, sem0),
            (srows1, drows1, sem1),
            (srows2, drows2, sem2))

    fire(0, *bufs[0])
    fire(_C, *bufs[1])

    def triple_body(t, carry):
        off0 = pl.multiple_of(t * 3 * _C, _C)
        for u in range(3):
            fire(off0 + (u + 2) * _C, *bufs[(u + 2) % 3])
            drain(*bufs[u])
            compute(off0 + u * _C, bufs[u][0], bufs[u][1])
        return carry

    lax.fori_loop(0, (_CHUNKS - 2) // 3, triple_body, 0)
    drain(*bufs[0])
    compute((_CHUNKS - 2) * _C, bufs[0][0], bufs[0][1])
    drain(*bufs[1])
    compute((_CHUNKS - 1) * _C, bufs[1][0], bufs[1][1])

    pltpu.sync_copy(obuf, out_hbm.at[pl.ds(base, _PER_W)])


@functools.partial(jax.jit, donate_argnums=())
def _edge_scores(hr, h, src, dst):
    mesh = plsc.VectorSubcoreMesh(core_axis_name="c", subcore_axis_name="s")
    k = pl.kernel(
        _edge_dot_body,
        out_type=jax.ShapeDtypeStruct((_E,), jnp.float32),
        mesh=mesh,
        compiler_params=pltpu.CompilerParams(needs_layout_passes=False),
        scratch_types=[
            pltpu.VMEM((_PER_W,), jnp.int32),
            pltpu.VMEM((_PER_W,), jnp.int32),
            pltpu.VMEM((_C, _D), jnp.float32),
            pltpu.VMEM((_C, _D), jnp.float32),
            pltpu.VMEM((_C, _D), jnp.float32),
            pltpu.VMEM((_C, _D), jnp.float32),
            pltpu.VMEM((_C, _D), jnp.float32),
            pltpu.VMEM((_C, _D), jnp.float32),
            pltpu.VMEM((16, 16), jnp.float32),
            pltpu.VMEM((_PER_W,), jnp.float32),
            pltpu.SemaphoreType.DMA,
            pltpu.SemaphoreType.DMA,
            pltpu.SemaphoreType.DMA,
        ],
    )
    return k(hr, h, src, dst)


def kernel(h, edge_index, r):
    hr = _weight_rows(h, r)
    src = edge_index[0]
    dst = edge_index[1]
    return _edge_scores(hr, h, src, dst)


# speedup vs baseline: 1.2170x; 1.2170x over previous
import functools

import jax
import jax.numpy as jnp
import numpy as np
from jax import lax
from jax.experimental import pallas as pl
from jax.experimental.pallas import tpu as pltpu
from jax.experimental.pallas import tpu_sc as plsc

_N = 10000
_D = 128
_E = 320000
_NC = 2
_NS = 16
_NW = _NC * _NS
_PER_W = _E // _NW
_C = 80
_CHUNKS = _PER_W // _C
_G = _C // 16
_K = _D // 16


def _hr_body(h_ref, r_ref, o_ref):
    o_ref[:, :] = h_ref[:, :] * r_ref[:, :]


def _weight_rows(h, r):
    return pl.pallas_call(
        _hr_body,
        out_shape=jax.ShapeDtypeStruct((_N, _D), jnp.float32),
    )(h, r.reshape(1, _D))


_BITREV = (0, 8, 4, 12, 2, 10, 6, 14, 1, 9, 5, 13, 3, 11, 7, 15)
_LANES = np.arange(16, dtype=np.int32)


def _edge_dot_body(hr_hbm, h_hbm, src_hbm, dst_hbm, out_hbm,
                   sidx, didx, srows0, drows0, srows1, drows1,
                   srows2, drows2, qbuf, obuf, sem0, sem1, sem2):
    wid = lax.axis_index("s") * _NC + lax.axis_index("c")
    base = wid * _PER_W
    pltpu.sync_copy(src_hbm.at[pl.ds(base, _PER_W)], sidx)
    pltpu.sync_copy(dst_hbm.at[pl.ds(base, _PER_W)], didx)

    def fire(off, sbuf, dbuf, sem):
        pltpu.async_copy(hr_hbm.at[sidx.at[pl.ds(off, _C)]], sbuf, sem)
        pltpu.async_copy(h_hbm.at[didx.at[pl.ds(off, _C)]], dbuf, sem)

    def drain(sbuf, dbuf, sem):
        pltpu.make_async_copy(hr_hbm.at[sidx.at[pl.ds(0, _C)]],
                              sbuf, sem).wait()
        pltpu.make_async_copy(h_hbm.at[didx.at[pl.ds(0, _C)]],
                              dbuf, sem).wait()

    lane = lax.iota(jnp.int32, 16)

    def compute(coff, sbuf, dbuf):
        def group_body(g, carry):
            e0 = g * 16
            for jj in range(0, 16, 2):
                ea = e0 + jj
                eb = e0 + jj + 1
                acc_a = sbuf[ea, pl.ds(0, 16)] * dbuf[ea, pl.ds(0, 16)]
                acc_b = sbuf[eb, pl.ds(0, 16)] * dbuf[eb, pl.ds(0, 16)]
                for k in range(1, _K):
                    acc_a = acc_a + (sbuf[ea, pl.ds(k * 16, 16)]
                                     * dbuf[ea, pl.ds(k * 16, 16)])
                    acc_b = acc_b + (sbuf[eb, pl.ds(k * 16, 16)]
                                     * dbuf[eb, pl.ds(k * 16, 16)])
                qbuf[jj, :] = acc_a
                qbuf[jj + 1, :] = acc_b
            s0 = plsc.load_gather(qbuf, [lane, jnp.zeros((16,), jnp.int32)])
            s1 = plsc.load_gather(qbuf, [lane, jnp.full((16,), 1, jnp.int32)])
            for l in range(2, 16, 2):
                s0 = s0 + plsc.load_gather(
                    qbuf, [lane, jnp.full((16,), l, jnp.int32)])
                s1 = s1 + plsc.load_gather(
                    qbuf, [lane, jnp.full((16,), l + 1, jnp.int32)])
            obuf[pl.ds(pl.multiple_of(coff + e0, 16), 16)] = s0 + s1
            return carry
        lax.fori_loop(0, _G, group_body, 0)

    bufs = ((srows0, drows0, sem0),
            (srows1, drows1, sem1),
            (srows2, drows2, sem2))

    fire(0, *bufs[0])
    fire(_C, *bufs[1])

    def triple_body(t, carry):
        off0 = pl.multiple_of(t * 3 * _C, _C)
        for u in range(3):
            fire(off0 + (u + 2) * _C, *bufs[(u + 2) % 3])
            drain(*bufs[u])
            compute(off0 + u * _C, bufs[u][0], bufs[u][1])
        return carry

    lax.fori_loop(0, (_CHUNKS - 2) // 3, triple_body, 0)
    drain(*bufs[0])
    compute((_CHUNKS - 2) * _C, bufs[0][0], bufs[0][1])
    drain(*bufs[1])
    compute((_CHUNKS - 1) * _C, bufs[1][0], bufs[1][1])

    pltpu.sync_copy(obuf, out_hbm.at[pl.ds(base, _PER_W)])


@functools.partial(jax.jit, donate_argnums=())
def _edge_scores(hr, h, src, dst):
    mesh = plsc.VectorSubcoreMesh(core_axis_name="c", subcore_axis_name="s")
    k = pl.kernel(
        _edge_dot_body,
        out_type=jax.ShapeDtypeStruct((_E,), jnp.float32),
        mesh=mesh,
        compiler_params=pltpu.CompilerParams(needs_layout_passes=False),
        scratch_types=[
            pltpu.VMEM((_PER_W,), jnp.int32),
            pltpu.VMEM((_PER_W,), jnp.int32),
            pltpu.VMEM((_C, _D), jnp.float32),
            pltpu.VMEM((_C, _D), jnp.float32),
            pltpu.VMEM((_C, _D), jnp.float32),
            pltpu.VMEM((_C, _D), jnp.float32),
            pltpu.VMEM((_C, _D), jnp.float32),
            pltpu.VMEM((_C, _D), jnp.float32),
            pltpu.VMEM((16, 16), jnp.float32),
            pltpu.VMEM((_PER_W,), jnp.float32),
            pltpu.SemaphoreType.DMA,
            pltpu.SemaphoreType.DMA,
            pltpu.SemaphoreType.DMA,
        ],
    )
    return k(hr, h, src, dst)


def kernel(h, edge_index, r):
    hr = _weight_rows(h, r)
    src = edge_index[0]
    dst = edge_index[1]
    return _edge_scores(hr, h, src, dst)
